# Initial kernel scaffold; baseline (speedup 1.0000x reference)
import jax
import jax.numpy as jnp
from jax.experimental import pallas as pl


def kernel(user_id, movie_title, user_table, movie_table):
    raise NotImplementedError("write your pallas kernel here")


# speedup vs baseline: 1.3622x; 1.3622x over previous
import functools

import jax
import jax.numpy as jnp
from jax import lax
from jax.experimental import pallas as pl
from jax.experimental.pallas import tpu as pltpu
from jax.experimental.pallas import tpu_sc as plsc

B = 16384
HIST = 50
D = 64

NC = 2
NS = 16
NW = NC * NS

U_PER_W = B // NW
CHUNK_U = 32
N_CHUNKS = U_PER_W // CHUNK_U
ROWS = CHUNK_U * HIST
IDX_MINOR = 100
IDX_MAJOR = ROWS // IDX_MINOR


def _sc_body(uid_hbm, midx_hbm, utab_hbm, mtab_hbm, out_hbm,
             midx_v, uidx_v, mrows_v, urows_v, out_v, sem):
    wid = lax.axis_index("s") * NC + lax.axis_index("c")
    wbase = wid * U_PER_W

    def chunk_body(g, _):
        ubase = wbase + g * CHUNK_U
        mrow0 = pl.multiple_of(ubase * HIST // IDX_MINOR, 8)
        pltpu.sync_copy(midx_hbm.at[pl.ds(mrow0, IDX_MAJOR)], midx_v)
        pltpu.sync_copy(uid_hbm.at[pl.ds(pl.multiple_of(ubase, 8), CHUNK_U)],
                        uidx_v)
        copies = []
        for j in range(IDX_MAJOR):
            copies.append(pltpu.async_copy(
                mtab_hbm.at[midx_v.at[j]],
                mrows_v.at[pl.ds(j * IDX_MINOR, IDX_MINOR)], sem))
        copies.append(pltpu.async_copy(utab_hbm.at[uidx_v], urows_v, sem))
        for c in copies:
            c.wait()

        lane = lax.iota(jnp.int32, 16)
        last_lane = lane == 15
        bfly = [lane ^ d for d in (8, 4, 2, 1)]

        def user_body(i, _):
            u0 = urows_v[i, pl.ds(0, 16)]
            u1 = urows_v[i, pl.ds(16, 16)]
            u2 = urows_v[i, pl.ds(32, 16)]
            u3 = urows_v[i, pl.ds(48, 16)]
            base_row = i * HIST
            for h in range(HIST):
                r = base_row + h
                p = (mrows_v[r, pl.ds(0, 16)] * u0
                     + mrows_v[r, pl.ds(16, 16)] * u1
                     + mrows_v[r, pl.ds(32, 16)] * u2
                     + mrows_v[r, pl.ds(48, 16)] * u3)
                for ix in bfly:
                    p = p + p.at[ix].get(mode="promise_in_bounds")
                plsc.store_scatter(out_v, [jnp.full((16,), r, jnp.int32)],
                                   p, mask=last_lane)
            return 0

        lax.fori_loop(0, CHUNK_U, user_body, 0)
        pltpu.sync_copy(out_v, out_hbm.at[pl.ds(ubase * HIST, ROWS)])
        return 0

    lax.fori_loop(0, N_CHUNKS, chunk_body, 0)


@jax.jit
def _run(uid_flat, midx_2d, user_table, movie_table):
    mesh = plsc.VectorSubcoreMesh(core_axis_name="c", subcore_axis_name="s")
    k = pl.kernel(
        _sc_body,
        out_type=jax.ShapeDtypeStruct((B * HIST,), jnp.float32),
        mesh=mesh,
        scratch_types=[
            pltpu.VMEM((IDX_MAJOR, IDX_MINOR), jnp.int32),
            pltpu.VMEM((CHUNK_U,), jnp.int32),
            pltpu.VMEM((ROWS, D), jnp.float32),
            pltpu.VMEM((CHUNK_U, D), jnp.float32),
            pltpu.VMEM((ROWS,), jnp.float32),
            pltpu.SemaphoreType.DMA,
        ],
        compiler_params=pltpu.CompilerParams(needs_layout_passes=False,
                                             use_tc_tiling_on_sc=False),
    )
    return k(uid_flat, midx_2d, user_table, movie_table)


def kernel(user_id, movie_title, user_table, movie_table):
    uid_flat = user_id.reshape(B)
    midx_2d = movie_title.reshape(B * HIST // IDX_MINOR, IDX_MINOR)
    out = _run(uid_flat, midx_2d, user_table, movie_table)
    return out.reshape(B, HIST)
